# Initial kernel scaffold; baseline (speedup 1.0000x reference)
import jax
import jax.numpy as jnp
from jax.experimental import pallas as pl


def kernel(x, Wr, br, W1, b1, W2, b2):
    raise NotImplementedError("write your pallas kernel here")


# speedup vs baseline: 1.4678x; 1.4678x over previous
import functools

import jax
import jax.numpy as jnp
from jax.experimental import pallas as pl
from jax.experimental.pallas import tpu as pltpu

T = 2048
D = 1024
FF = 4096
E = 8
TOP_K = 2

RBLK = 256
TBLK = 256
FBLK = 1024


def _router_body(x_ref, wr_ref, br_ref, wdense_ref):
    x = x_ref[...]
    logits = jnp.dot(x, wr_ref[...], preferred_element_type=jnp.float32)
    logits = logits + br_ref[...]
    lanes = jax.lax.broadcasted_iota(jnp.int32, logits.shape, 1)
    m1 = jnp.max(logits, axis=1, keepdims=True)
    am1 = jnp.argmax(logits, axis=1)[:, None]
    neg = jnp.full_like(logits, -jnp.inf)
    l2 = jnp.where(lanes == am1, neg, logits)
    m2 = jnp.max(l2, axis=1, keepdims=True)
    am2 = jnp.argmax(l2, axis=1)[:, None]
    z = jnp.exp(m2 - m1)
    w1 = 1.0 / (1.0 + z)
    w2 = z / (1.0 + z)
    wd = jnp.where(lanes == am1, w1, 0.0) + jnp.where(lanes == am2, w2, 0.0)
    wdense_ref[...] = wd


def _router(x, Wr, br):
    return pl.pallas_call(
        _router_body,
        grid=(T // RBLK,),
        in_specs=[
            pl.BlockSpec((RBLK, D), lambda i: (i, 0)),
            pl.BlockSpec((D, E), lambda i: (0, 0)),
            pl.BlockSpec((1, E), lambda i: (0, 0)),
        ],
        out_specs=pl.BlockSpec((RBLK, E), lambda i: (i, 0)),
        out_shape=jax.ShapeDtypeStruct((T, E), jnp.float32),
    )(x, Wr, br.reshape(1, E))


def _ffn_body(x_ref, w1_ref, b1_ref, w2_ref, b2_ref, wd_ref, out_ref):
    e = pl.program_id(1)
    f = pl.program_id(2)

    @pl.when((e == 0) & (f == 0))
    def _():
        out_ref[...] = jnp.zeros_like(out_ref)

    x = x_ref[...]
    h = jnp.dot(x, w1_ref[0], preferred_element_type=jnp.float32) + b1_ref[0]
    h = 0.5 * h * (1.0 + jax.lax.erf(h * 0.7071067811865476))
    y = jnp.dot(h, w2_ref[0], preferred_element_type=jnp.float32)
    wd = wd_ref[...]
    lanes = jax.lax.broadcasted_iota(jnp.int32, wd.shape, 1)
    w = jnp.sum(jnp.where(lanes == e, wd, 0.0), axis=1, keepdims=True)

    @pl.when(f == 0)
    def _():
        out_ref[...] += w * (y + b2_ref[0])

    @pl.when(f != 0)
    def _():
        out_ref[...] += w * y


def _ffn(x, W1, b1, W2, b2, wdense):
    return pl.pallas_call(
        _ffn_body,
        grid=(T // TBLK, E, FF // FBLK),
        in_specs=[
            pl.BlockSpec((TBLK, D), lambda i, e, f: (i, 0)),
            pl.BlockSpec((1, D, FBLK), lambda i, e, f: (e, 0, f)),
            pl.BlockSpec((1, 1, FBLK), lambda i, e, f: (e, 0, f)),
            pl.BlockSpec((1, FBLK, D), lambda i, e, f: (e, f, 0)),
            pl.BlockSpec((1, 1, D), lambda i, e, f: (e, 0, 0)),
            pl.BlockSpec((TBLK, E), lambda i, e, f: (i, 0)),
        ],
        out_specs=pl.BlockSpec((TBLK, D), lambda i, e, f: (i, 0)),
        out_shape=jax.ShapeDtypeStruct((T, D), jnp.float32),
    )(x, W1, b1.reshape(E, 1, FF), W2, b2.reshape(E, 1, D), wdense)


def kernel(x, Wr, br, W1, b1, W2, b2):
    wdense = _router(x, Wr, br)
    return _ffn(x, W1, b1, W2, b2, wdense)


# speedup vs baseline: 2.1689x; 1.4777x over previous
import functools

import jax
import jax.numpy as jnp
from jax import lax
from jax.experimental import pallas as pl
from jax.experimental.pallas import tpu as pltpu
from jax.experimental.pallas import tpu_sc as plsc

T = 2048
D = 1024
FF = 4096
E = 8
TOP_K = 2

A = T * TOP_K
TBLK = 128
P = A + E * TBLK
NBLK = P // TBLK
NBLK_PAD = 48

RBLK = 256
L = 16

_SQRT1_2 = 0.7071067811865476

_SC_PARAMS = pltpu.CompilerParams(
    use_tc_tiling_on_sc=False, needs_layout_passes=False)


def _router_body(x_ref, wr_ref, br_ref, eidx_ref, ew_ref, xcopy_ref):
    x = x_ref[...]
    xcopy_ref[...] = x
    logits = jnp.dot(x, wr_ref[...], preferred_element_type=jnp.float32)
    logits = logits + br_ref[...]
    lanes = lax.broadcasted_iota(jnp.int32, logits.shape, 1)
    m1 = jnp.max(logits, axis=1, keepdims=True)
    am1 = jnp.argmax(logits, axis=1)[:, None]
    neg = jnp.full_like(logits, -jnp.inf)
    l2 = jnp.where(lanes == am1, neg, logits)
    m2 = jnp.max(l2, axis=1, keepdims=True)
    am2 = jnp.argmax(l2, axis=1)[:, None]
    z = jnp.exp(m2 - m1)
    w1 = 1.0 / (1.0 + z)
    w2 = z / (1.0 + z)
    eidx_ref[...] = jnp.concatenate([am1.astype(jnp.int32), am2.astype(jnp.int32)], axis=1)
    ew_ref[...] = jnp.concatenate([w1, w2], axis=1)


def _router(x, Wr, br):
    return pl.pallas_call(
        _router_body,
        grid=(T // RBLK,),
        in_specs=[
            pl.BlockSpec((RBLK, D), lambda i: (i, 0)),
            pl.BlockSpec((D, E), lambda i: (0, 0)),
            pl.BlockSpec((1, E), lambda i: (0, 0)),
        ],
        out_specs=[
            pl.BlockSpec((RBLK, TOP_K), lambda i: (i, 0)),
            pl.BlockSpec((RBLK, TOP_K), lambda i: (i, 0)),
            pl.BlockSpec((RBLK, D), lambda i: (i, 0)),
        ],
        out_shape=[
            jax.ShapeDtypeStruct((T, TOP_K), jnp.int32),
            jax.ShapeDtypeStruct((T, TOP_K), jnp.float32),
            jax.ShapeDtypeStruct((T, D), jnp.float32),
        ],
    )(x, Wr, br.reshape(1, E))


def _dispatch_body(eid_hbm, w_hbm, x_hbm, roww_hbm, posflat_hbm, blkexp_hbm, xs_hbm,
                   eid_v, w_v, rowtok_v, roww_v, posflat_v, blkexp_v, base_v, cnt_v,
                   idxg_v, rowsA, rowsB, gsA, gsB, osA, osB, rowtok_s):
    c = lax.axis_index("c")
    s = lax.axis_index("s")

    @pl.when(s == 0)
    def _():
        lane = lax.iota(jnp.int32, L)
        pltpu.sync_copy(eid_hbm, eid_v)
        pltpu.sync_copy(w_hbm, w_v)

        zero = jnp.zeros((L,), jnp.int32)

        def z_body(i, _):
            rowtok_v[pl.ds(i * L, L)] = jnp.zeros((L,), jnp.int32)
            roww_v[pl.ds(i * L, L)] = jnp.zeros((L,), jnp.float32)
            return 0

        lax.fori_loop(0, P // L, z_body, 0)

        def c_body(j, cnts):
            v = eid_v[pl.ds(j * L, L)]
            for e in range(E):
                pc = jnp.sum((v == e).astype(jnp.int32))
                cnts = cnts + jnp.where(lane == e, jnp.full((L,), pc), zero)
            return cnts

        cnts = lax.fori_loop(0, A // L, c_body, jnp.zeros((L,), jnp.int32))

        pad = ((cnts + (TBLK - 1)) // TBLK) * TBLK
        csum = plsc.cumsum(pad)
        base = csum - pad
        base_v[...] = base

        one = jnp.ones((L,), jnp.int32)
        for cb in range(NBLK_PAD // L):
            sblk = (lane + (cb * L)) * jnp.full((L,), TBLK)
            be = jnp.zeros((L,), jnp.int32)
            for e in range(1, E):
                base_e = plsc.load_gather(base_v, [jnp.full((L,), e, jnp.int32)])
                be = be + jnp.where(sblk >= base_e, one, zero)
            blkexp_v[pl.ds(cb * L, L)] = be

        def p2_body(j, cnts):
            cnt_v[...] = cnts
            v = eid_v[pl.ds(j * L, L)]
            wv = w_v[pl.ds(j * L, L)]
            basev = plsc.load_gather(base_v, [v])
            cntv = plsc.load_gather(cnt_v, [v])
            within = jnp.zeros((L,), jnp.int32)
            delta = jnp.zeros((L,), jnp.int32)
            one = jnp.ones((L,), jnp.int32)
            for e in range(E):
                m = v == e
                cs = plsc.cumsum(m.astype(jnp.int32))
                within = jnp.where(m, cs - one, within)
                pc = jnp.sum(m.astype(jnp.int32))
                delta = delta + jnp.where(lane == e, jnp.full((L,), pc), zero)
            pos = basev + cntv + within
            aidx = lane + jnp.full((L,), j * L)
            tok = lax.shift_right_logical(aidx, one)
            plsc.store_scatter(rowtok_v, [pos], tok)
            plsc.store_scatter(roww_v, [pos], wv)
            slotbase = (aidx & one) * jnp.full((L,), T)
            plsc.store_scatter(posflat_v, [slotbase + tok], pos)
            return cnts + delta

        lax.fori_loop(0, A // L, p2_body, jnp.zeros((L,), jnp.int32))

        pltpu.sync_copy(rowtok_v, rowtok_s)

        @pl.when(c == 0)
        def _():
            pltpu.sync_copy(roww_v, roww_hbm)
            pltpu.sync_copy(posflat_v, posflat_hbm)
            pltpu.sync_copy(blkexp_v, blkexp_hbm)

    plsc.subcore_barrier()

    wid = s * 2 + c
    rpw = P // 32
    chunk = rpw // 4
    base_r = wid * rpw
    pltpu.sync_copy(rowtok_s.at[pl.ds(base_r, rpw)], idxg_v)
    bufs = (rowsA, rowsB)
    gsems = (gsA, gsB)
    osems = (osA, osB)
    gcp = {}
    ocp = {}
    gcp[0] = pltpu.async_copy(x_hbm.at[idxg_v.at[pl.ds(0, chunk)]], bufs[0], gsems[0])
    for cc in range(4):
        cur = cc % 2
        gcp[cc].wait()
        if cc + 1 < 4:
            if cc - 1 >= 0:
                ocp[cc - 1].wait()
            nxt = (cc + 1) % 2
            gcp[cc + 1] = pltpu.async_copy(
                x_hbm.at[idxg_v.at[pl.ds((cc + 1) * chunk, chunk)]],
                bufs[nxt], gsems[nxt])
        ocp[cc] = pltpu.async_copy(
            bufs[cur], xs_hbm.at[pl.ds(base_r + cc * chunk, chunk)], osems[cur])
    ocp[2].wait()
    ocp[3].wait()


def _dispatch_gather(eid_flat, w_flat, x):
    mesh = plsc.VectorSubcoreMesh(core_axis_name="c", subcore_axis_name="s")
    rpw = P // 32
    f = functools.partial(
        pl.kernel,
        out_type=(
            jax.ShapeDtypeStruct((P,), jnp.float32),
            jax.ShapeDtypeStruct((A,), jnp.int32),
            jax.ShapeDtypeStruct((NBLK_PAD,), jnp.int32),
            jax.ShapeDtypeStruct((P, D), jnp.float32),
        ),
        mesh=mesh,
        compiler_params=_SC_PARAMS,
        scratch_types=[
            pltpu.VMEM((A,), jnp.int32),
            pltpu.VMEM((A,), jnp.float32),
            pltpu.VMEM((P,), jnp.int32),
            pltpu.VMEM((P,), jnp.float32),
            pltpu.VMEM((A,), jnp.int32),
            pltpu.VMEM((NBLK_PAD,), jnp.int32),
            pltpu.VMEM((L,), jnp.int32),
            pltpu.VMEM((L,), jnp.int32),
            pltpu.VMEM((rpw,), jnp.int32),
            pltpu.VMEM((rpw // 4, D), jnp.float32),
            pltpu.VMEM((rpw // 4, D), jnp.float32),
            pltpu.SemaphoreType.DMA,
            pltpu.SemaphoreType.DMA,
            pltpu.SemaphoreType.DMA,
            pltpu.SemaphoreType.DMA,
            pltpu.VMEM_SHARED((P,), jnp.int32),
        ],
    )(_dispatch_body)
    return f(eid_flat, w_flat, x)


def _make_gather_body(nrows, nchunks):
    rpw = nrows // 32
    chunk = rpw // nchunks

    def body(tab_hbm, idx_hbm, out_hbm, idx_v, rowsA, rowsB, gsA, gsB, osA, osB):
        wid = lax.axis_index("s") * 2 + lax.axis_index("c")
        base = wid * rpw
        pltpu.sync_copy(idx_hbm.at[pl.ds(base, rpw)], idx_v)
        bufs = (rowsA, rowsB)
        gsems = (gsA, gsB)
        osems = (osA, osB)
        gcp = {}
        ocp = {}
        gcp[0] = pltpu.async_copy(
            tab_hbm.at[idx_v.at[pl.ds(0, chunk)]], bufs[0], gsems[0])
        for c in range(nchunks):
            cur = c % 2
            gcp[c].wait()
            if c + 1 < nchunks:
                if c - 1 >= 0:
                    ocp[c - 1].wait()
                nxt = (c + 1) % 2
                gcp[c + 1] = pltpu.async_copy(
                    tab_hbm.at[idx_v.at[pl.ds((c + 1) * chunk, chunk)]],
                    bufs[nxt], gsems[nxt])
            ocp[c] = pltpu.async_copy(
                bufs[cur], out_hbm.at[pl.ds(base + c * chunk, chunk)], osems[cur])
        ocp[nchunks - 2].wait()
        ocp[nchunks - 1].wait()

    return body, rpw, chunk


def _sc_row_gather(table, idx, nrows, nchunks):
    body, rpw, chunk = _make_gather_body(nrows, nchunks)
    mesh = plsc.VectorSubcoreMesh(core_axis_name="c", subcore_axis_name="s")
    f = functools.partial(
        pl.kernel,
        out_type=jax.ShapeDtypeStruct((nrows, D), jnp.float32),
        mesh=mesh,
        compiler_params=_SC_PARAMS,
        scratch_types=[
            pltpu.VMEM((rpw,), jnp.int32),
            pltpu.VMEM((chunk, D), jnp.float32),
            pltpu.VMEM((chunk, D), jnp.float32),
            pltpu.SemaphoreType.DMA,
            pltpu.SemaphoreType.DMA,
            pltpu.SemaphoreType.DMA,
            pltpu.SemaphoreType.DMA,
        ],
    )(body)
    return f(table, idx)


def _combine_gather(ys, posflat):
    return _sc_row_gather(ys, posflat, A, 4)


def _cast_body(w1_ref, w2_ref, o1_ref, o2_ref):
    o1_ref[...] = w1_ref[...].astype(jnp.bfloat16)
    o2_ref[...] = w2_ref[...].astype(jnp.bfloat16)


_CFB = 2048


def _cast_weights(W1, W2):
    return pl.pallas_call(
        _cast_body,
        grid=(E, FF // _CFB),
        in_specs=[
            pl.BlockSpec((1, D, _CFB), lambda e, f: (e, 0, f)),
            pl.BlockSpec((1, _CFB, D), lambda e, f: (e, f, 0)),
        ],
        out_specs=[
            pl.BlockSpec((1, D, _CFB), lambda e, f: (e, 0, f)),
            pl.BlockSpec((1, _CFB, D), lambda e, f: (e, f, 0)),
        ],
        out_shape=[
            jax.ShapeDtypeStruct((E, D, FF), jnp.bfloat16),
            jax.ShapeDtypeStruct((E, FF, D), jnp.bfloat16),
        ],
    )(W1, W2)


def _ffn_body(be_ref, xs_ref, rw_ref, w1_ref, b1_ref, w2_ref, b2_ref, out_ref):
    xb = xs_ref[...].astype(jnp.bfloat16)
    h = jnp.dot(xb, w1_ref[0], preferred_element_type=jnp.float32) + b1_ref[0]
    h = 0.5 * h * (1.0 + lax.erf(h * _SQRT1_2))
    y = jnp.dot(h.astype(jnp.bfloat16), w2_ref[0], preferred_element_type=jnp.float32) + b2_ref[0]
    out_ref[...] = y * rw_ref[...]


def _grouped_ffn(blkexp, xs, roww, W1, b1, W2, b2):
    grid_spec = pltpu.PrefetchScalarGridSpec(
        num_scalar_prefetch=1,
        grid=(NBLK,),
        in_specs=[
            pl.BlockSpec((TBLK, D), lambda i, be: (i, 0)),
            pl.BlockSpec((TBLK, 1), lambda i, be: (i, 0)),
            pl.BlockSpec((1, D, FF), lambda i, be: (be[i], 0, 0)),
            pl.BlockSpec((1, 1, FF), lambda i, be: (be[i], 0, 0)),
            pl.BlockSpec((1, FF, D), lambda i, be: (be[i], 0, 0)),
            pl.BlockSpec((1, 1, D), lambda i, be: (be[i], 0, 0)),
        ],
        out_specs=pl.BlockSpec((TBLK, D), lambda i, be: (i, 0)),
    )
    return pl.pallas_call(
        _ffn_body,
        grid_spec=grid_spec,
        out_shape=jax.ShapeDtypeStruct((P, D), jnp.float32),
    )(blkexp, xs, roww.reshape(P, 1), W1, b1.reshape(E, 1, FF), W2,
      b2.reshape(E, 1, D))


def _add_body(a_ref, b_ref, out_ref):
    out_ref[...] = a_ref[...] + b_ref[...]


def _pair_add(yg):
    nb = T // RBLK
    return pl.pallas_call(
        _add_body,
        grid=(nb,),
        in_specs=[
            pl.BlockSpec((RBLK, D), lambda i: (i, 0)),
            pl.BlockSpec((RBLK, D), lambda i, _nb=nb: (i + _nb, 0)),
        ],
        out_specs=pl.BlockSpec((RBLK, D), lambda i: (i, 0)),
        out_shape=jax.ShapeDtypeStruct((T, D), jnp.float32),
    )(yg, yg)


def kernel(x, Wr, br, W1, b1, W2, b2):
    eidx, ew, xcopy = _router(x, Wr, br)
    roww, posflat, blkexp, xs = _dispatch_gather(
        eidx.reshape(A), ew.reshape(A), xcopy)
    W1b, W2b = _cast_weights(W1, W2)
    ys = _grouped_ffn(blkexp, xs, roww, W1b, b1, W2b, b2)
    yg = _combine_gather(ys, posflat)
    return _pair_add(yg)
